# baseline (speedup 1.0000x reference)
import functools

import jax
import jax.numpy as jnp
from jax import lax
from jax.experimental import pallas as pl
from jax.experimental.pallas import tpu as pltpu
from jax.experimental.pallas import tpu_sc as plsc

L = 16
NW = 32
B = 16384
D = 64
BPW = B // NW
V = 1000000
BLK = 128
NBLK = (V + BLK - 1) // BLK
PHYS_COLS = NBLK * BLK
RB = 3
RW = RB * BLK
NR = 82
NBUF = 4
LAST_BASE = PHYS_COLS - RW
SR = 16
WLSZ = B + L
SENT = 1 << 29

_mesh = plsc.VectorSubcoreMesh(core_axis_name="c", subcore_axis_name="s")


def _iota():
    return lax.broadcasted_iota(jnp.int32, (L,), 0)


def _lane(vec, lane):
    idx = jnp.full((L,), lane, jnp.int32)
    dnums = lax.GatherDimensionNumbers(
        offset_dims=(), collapsed_slice_dims=(0,), start_index_map=(0,))
    g = lax.gather(vec, idx[:, None], dnums, (1,),
                   mode=lax.GatherScatterMode.PROMISE_IN_BOUNDS)
    return g[0]


@functools.partial(
    pl.kernel,
    out_type=(
        jax.ShapeDtypeStruct((B * D,), jnp.float32),
        jax.ShapeDtypeStruct((B * D,), jnp.float32),
    ),
    mesh=_mesh,
    compiler_params=pltpu.CompilerParams(needs_layout_passes=False),
    scratch_types=[
        pltpu.VMEM((2048,), jnp.int32),
        pltpu.VMEM((WLSZ,), jnp.int32),
        pltpu.VMEM((D, RW), jnp.float32),
        pltpu.VMEM((D, RW), jnp.float32),
        pltpu.VMEM((D, RW), jnp.float32),
        pltpu.VMEM((D, RW), jnp.float32),
        pltpu.VMEM((SR, D), jnp.float32),
        pltpu.SemaphoreType.DMA,
        pltpu.SemaphoreType.DMA,
        pltpu.SemaphoreType.DMA,
        pltpu.SemaphoreType.DMA,
        pltpu.SemaphoreType.DMA,
        pltpu.SemaphoreType.DMA,
    ],
)
def _mf_stream_extract(users_hbm, items_hbm, utT_hbm, itT_hbm,
                       uflat_hbm, iflat_hbm,
                       chunk, wl, buf0, buf1, buf2, buf3, stage,
                       sem0, sem1, sem2, sem3, semX, semC):
    bufs = (buf0, buf1, buf2, buf3)
    sems = (sem0, sem1, sem2, sem3)

    def drain_one(i, carry):
        pltpu.make_async_copy(
            uflat_hbm.at[pl.ds(0, D)], stage.at[0], semX).wait()
        return carry
    wid = lax.axis_index("s") * 2 + lax.axis_index("c")
    c0 = 244 * wid + jnp.minimum(wid, 5)
    nb = jnp.where(wid < 5, 245, 244)
    lo_k = c0 * BLK
    hi_k = (c0 + nb) * BLK

    def build_wl(src_hbm):
        def chunk_body(s, cnt):
            pltpu.async_copy(src_hbm.at[pl.ds(s * 2048, 2048)], chunk,
                             semC).wait()
            def vec_body(v, cnt):
                kv = chunk[pl.ds(v * L, L)]
                bv = (s * 2048 + v * L) + _iota()
                m = (kv >= lo_k) & (kv < hi_k)
                pack = ((kv - lo_k) << 14) | bv
                cs = plsc.cumsum(m.astype(jnp.int32))
                pos = jnp.maximum(cnt + cs - 1, 0)
                plsc.store_scatter(wl, [pos], pack, mask=m)
                return cnt + cs[L - 1]
            return lax.fori_loop(0, 2048 // L, vec_body, cnt)
        cnt = lax.fori_loop(0, B // 2048, chunk_body, jnp.int32(0))
        plsc.store_scatter(wl, [cnt + _iota()],
                           jnp.full((L,), SENT, jnp.int32),
                           mask=jnp.full((L,), True))
        return cnt

    def issue(src_hbm, r, buf, sem):
        base_k = jnp.minimum((c0 + RB * r) * BLK, LAST_BASE)
        base_k = pl.multiple_of(base_k, BLK)
        return pltpu.async_copy(src_hbm.at[:, pl.ds(base_k, RW)], buf, sem)

    def pass_table(src_hbm, dst_hbm, cnt):
        nv = (cnt + L - 1) // L

        def process_round(r, buf, sem, h):
            pltpu.make_async_copy(src_hbm.at[:, pl.ds(0, RW)], buf, sem).wait()
            r_lo = (c0 + RB * r) * BLK
            r_hi = jnp.minimum(r_lo + RW, hi_k)
            base_k = jnp.minimum(r_lo, LAST_BASE)

            def scan_j(j, h):
                wv = wl[pl.ds(j * L, L)]
                kg = (wv >> 14) + lo_k
                m = (kg >= r_lo) & (kg < r_hi)

                def hit_cond(state):
                    m, _ = state
                    return jnp.any(m)

                def hit_body(state):
                    m, h = state
                    l_vec = plsc.all_reduce_ffs(m)
                    pk = _lane(wv, l_vec[0])
                    kg_s = (pk >> 14) + lo_k
                    b_s = pk & (B - 1)
                    colw = kg_s - base_k
                    hmod = h % SR
                    cvec = jnp.full((L,), colw, jnp.int32)
                    for dc in range(D // L):
                        g = plsc.load_gather(
                            buf, [dc * L + _iota(), cvec])
                        stage[hmod, pl.ds(dc * L, L)] = g
                    off = pl.multiple_of(b_s * D, 8)
                    pltpu.async_copy(stage.at[hmod],
                                     dst_hbm.at[pl.ds(off, D)], semX)
                    @pl.when((h + 1) % SR == 0)
                    def _():
                        lax.fori_loop(0, SR, drain_one, 0)
                    return m & (_iota() != l_vec), h + 1

                m, h = lax.while_loop(hit_cond, hit_body, (m, h))
                return h

            return lax.fori_loop(0, nv, scan_j, h)

        for par in range(NBUF):
            issue(src_hbm, par, bufs[par], sems[par])

        def group_body(rr, h):
            for par in range(NBUF):
                r = rr * NBUF + par
                h = lax.cond(
                    r < NR,
                    lambda h, r=r, par=par: process_round(
                        r, bufs[par], sems[par], h),
                    lambda h: h,
                    h,
                )

                @pl.when(r + NBUF < NR)
                def _(r=r, par=par):
                    issue(src_hbm, r + NBUF, bufs[par], sems[par])
            return h

        h = lax.fori_loop(0, (NR + NBUF - 1) // NBUF, group_body, jnp.int32(0))
        lax.fori_loop(0, h % SR, drain_one, 0)

    cnt_u = build_wl(users_hbm)
    pass_table(utT_hbm, uflat_hbm, cnt_u)
    cnt_i = build_wl(items_hbm)
    pass_table(itT_hbm, iflat_hbm, cnt_i)


@functools.partial(
    pl.kernel,
    out_type=jax.ShapeDtypeStruct((B,), jnp.float32),
    mesh=_mesh,
    compiler_params=pltpu.CompilerParams(needs_layout_passes=False),
    scratch_types=[
        pltpu.VMEM((BPW * D,), jnp.float32),
        pltpu.VMEM((BPW * D,), jnp.float32),
        pltpu.VMEM((BPW,), jnp.float32),
        pltpu.SemaphoreType.DMA,
        pltpu.SemaphoreType.DMA,
    ],
)
def _mf_dot(uflat_hbm, iflat_hbm, out_hbm, uv, iv, out_v, semU, semI):
    wid = lax.axis_index("s") * 2 + lax.axis_index("c")
    base = wid * BPW
    cu = pltpu.async_copy(uflat_hbm.at[pl.ds(base * D, BPW * D)], uv, semU)
    ci = pltpu.async_copy(iflat_hbm.at[pl.ds(base * D, BPW * D)], iv, semI)
    cu.wait()
    ci.wait()
    for b0 in range(BPW // L):
        acc = jnp.zeros((L,), jnp.float32)
        row = (b0 * L + _iota()) * D

        def body(d, acc):
            idx = row + d
            u = plsc.load_gather(uv, [idx])
            i = plsc.load_gather(iv, [idx])
            return acc + u * i

        acc = lax.fori_loop(0, D, body, acc)
        out_v[pl.ds(b0 * L, L)] = acc
    pltpu.sync_copy(out_v, out_hbm.at[pl.ds(base, BPW)])


def kernel(users, items, user_table, item_table):
    u = users.astype(jnp.int32)
    i = items.astype(jnp.int32)
    u_flat, i_flat = _mf_stream_extract(u, i, user_table.T, item_table.T)
    return _mf_dot(u_flat, i_flat)


# speedup vs baseline: 1.3641x; 1.3641x over previous
import functools

import jax
import jax.numpy as jnp
from jax import lax
from jax.experimental import pallas as pl
from jax.experimental.pallas import tpu as pltpu
from jax.experimental.pallas import tpu_sc as plsc

L = 16
NW = 32
B = 16384
D = 64
BPW = B // NW
V = 1000000
BLK = 128
NBLK = (V + BLK - 1) // BLK
PHYS_COLS = NBLK * BLK
RB = 2
RW = RB * BLK
RSH = 8
NR = 123
NRP = 128
NBUF = 4
LAST_BASE = PHYS_COLS - RW
SR = 16
WLSZ = B + L
SENT = 1 << 29

_mesh = plsc.VectorSubcoreMesh(core_axis_name="c", subcore_axis_name="s")


def _iota():
    return lax.broadcasted_iota(jnp.int32, (L,), 0)


def _lane(vec, lane):
    idx = jnp.full((L,), lane, jnp.int32)
    dnums = lax.GatherDimensionNumbers(
        offset_dims=(), collapsed_slice_dims=(0,), start_index_map=(0,))
    g = lax.gather(vec, idx[:, None], dnums, (1,),
                   mode=lax.GatherScatterMode.PROMISE_IN_BOUNDS)
    return g[0]


@functools.partial(
    pl.kernel,
    out_type=(
        jax.ShapeDtypeStruct((B * D,), jnp.float32),
        jax.ShapeDtypeStruct((B * D,), jnp.float32),
    ),
    mesh=_mesh,
    compiler_params=pltpu.CompilerParams(needs_layout_passes=False),
    scratch_types=[
        pltpu.VMEM((2048,), jnp.int32),
        pltpu.VMEM((WLSZ,), jnp.int32),
        pltpu.VMEM((WLSZ,), jnp.int32),
        pltpu.VMEM((NRP,), jnp.int32),
        pltpu.VMEM((NRP,), jnp.int32),
        pltpu.VMEM((NRP,), jnp.int32),
        pltpu.VMEM((D, RW), jnp.float32),
        pltpu.VMEM((D, RW), jnp.float32),
        pltpu.VMEM((D, RW), jnp.float32),
        pltpu.VMEM((D, RW), jnp.float32),
        pltpu.VMEM((SR, D), jnp.float32),
        pltpu.SemaphoreType.DMA,
        pltpu.SemaphoreType.DMA,
        pltpu.SemaphoreType.DMA,
        pltpu.SemaphoreType.DMA,
        pltpu.SemaphoreType.DMA,
        pltpu.SemaphoreType.DMA,
    ],
)
def _mf_stream_extract(users_hbm, items_hbm, utT_hbm, itT_hbm,
                       uflat_hbm, iflat_hbm,
                       chunk, wl, wl2, counts, starts, cur,
                       buf0, buf1, buf2, buf3, stage,
                       sem0, sem1, sem2, sem3, semX, semC):
    bufs = (buf0, buf1, buf2, buf3)
    sems = (sem0, sem1, sem2, sem3)

    def drain_one(i, carry):
        pltpu.make_async_copy(
            uflat_hbm.at[pl.ds(0, D)], stage.at[0], semX).wait()
        return carry
    wid = lax.axis_index("s") * 2 + lax.axis_index("c")
    c0 = 244 * wid + jnp.minimum(wid, 5)
    nb = jnp.where(wid < 5, 245, 244)
    lo_k = c0 * BLK
    hi_k = (c0 + nb) * BLK

    def build_wl(src_hbm):
        for i in range(NRP // L):
            counts[pl.ds(i * L, L)] = jnp.zeros((L,), jnp.int32)

        def chunk_body(s, cnt):
            pltpu.async_copy(src_hbm.at[pl.ds(s * 2048, 2048)], chunk,
                             semC).wait()
            def vec_body(v, cnt):
                kv = chunk[pl.ds(v * L, L)]
                bv = (s * 2048 + v * L) + _iota()
                m = (kv >= lo_k) & (kv < hi_k)
                pack = ((kv - lo_k) << 14) | bv
                cs = plsc.cumsum(m.astype(jnp.int32))
                pos = jnp.maximum(cnt + cs - 1, 0)
                plsc.store_scatter(wl, [pos], pack, mask=m)
                rv = jnp.clip((kv - lo_k) >> RSH, 0, NRP - 1)
                plsc.addupdate_scatter(counts, [rv],
                                       jnp.ones((L,), jnp.int32), mask=m)
                return cnt + cs[L - 1]
            return lax.fori_loop(0, 2048 // L, vec_body, cnt)
        return lax.fori_loop(0, B // 2048, chunk_body, jnp.int32(0))

    def bucketize(cnt):
        carry = jnp.int32(0)
        for i in range(NRP // L):
            c = counts[pl.ds(i * L, L)]
            cs = plsc.cumsum(c)
            sv = cs - c + carry
            starts[pl.ds(i * L, L)] = sv
            cur[pl.ds(i * L, L)] = sv
            carry = carry + cs[L - 1]

        def place(j, _):
            mv = (j * L + _iota()) < cnt
            wv = wl[pl.ds(j * L, L)]
            rv = jnp.clip((wv >> 14) >> RSH, 0, NRP - 1)
            dup, _last = plsc.scan_count(rv, mask=mv)
            basev = plsc.load_gather(cur, [rv])
            plsc.store_scatter(wl2, [basev + dup - 1], wv, mask=mv)
            plsc.addupdate_scatter(cur, [rv],
                                   jnp.ones((L,), jnp.int32), mask=mv)
            return 0
        lax.fori_loop(0, (cnt + L - 1) // L, place, 0)

    def issue(src_hbm, r, buf, sem):
        base_k = jnp.minimum((c0 + RB * r) * BLK, LAST_BASE)
        base_k = pl.multiple_of(base_k, BLK)
        return pltpu.async_copy(src_hbm.at[:, pl.ds(base_k, RW)], buf, sem)

    def pass_table(src_hbm, dst_hbm, cnt):
        def process_round(r, buf, sem, h):
            pltpu.make_async_copy(src_hbm.at[:, pl.ds(0, RW)], buf, sem).wait()
            r_lo = (c0 + RB * r) * BLK
            base_k = jnp.minimum(r_lo, LAST_BASE)
            g = plsc.load_gather(starts, [r + jnp.minimum(_iota(), 1)])
            s_r = g[0]
            e_r = g[1]

            def seg_cond(state):
                t, _ = state
                return t < e_r

            def seg_body(state):
                t, h = state
                wv = plsc.load_gather(wl2, [t + _iota()])
                m = (t + _iota()) < e_r

                def hit_cond(state):
                    m, _ = state
                    return jnp.any(m)

                def hit_body(state):
                    m, h = state
                    l_vec = plsc.all_reduce_ffs(m)
                    pk = _lane(wv, l_vec[0])
                    kg_s = (pk >> 14) + lo_k
                    b_s = pk & (B - 1)
                    colw = kg_s - base_k
                    hmod = h % SR
                    cvec = jnp.full((L,), colw, jnp.int32)
                    for dc in range(D // L):
                        g = plsc.load_gather(
                            buf, [dc * L + _iota(), cvec])
                        stage[hmod, pl.ds(dc * L, L)] = g
                    off = pl.multiple_of(b_s * D, 8)
                    pltpu.async_copy(stage.at[hmod],
                                     dst_hbm.at[pl.ds(off, D)], semX)
                    @pl.when((h + 1) % SR == 0)
                    def _():
                        lax.fori_loop(0, SR, drain_one, 0)
                    return m & (_iota() != l_vec), h + 1

                m, h = lax.while_loop(hit_cond, hit_body, (m, h))
                return t + L, h

            _, h = lax.while_loop(seg_cond, seg_body, (s_r, h))
            return h

        for par in range(NBUF):
            issue(src_hbm, par, bufs[par], sems[par])

        def group_body(rr, h):
            for par in range(NBUF):
                r = rr * NBUF + par
                h = lax.cond(
                    r < NR,
                    lambda h, r=r, par=par: process_round(
                        r, bufs[par], sems[par], h),
                    lambda h: h,
                    h,
                )

                @pl.when(r + NBUF < NR)
                def _(r=r, par=par):
                    issue(src_hbm, r + NBUF, bufs[par], sems[par])
            return h

        h = lax.fori_loop(0, (NR + NBUF - 1) // NBUF, group_body, jnp.int32(0))
        lax.fori_loop(0, h % SR, drain_one, 0)

    cnt_u = build_wl(users_hbm)
    bucketize(cnt_u)
    pass_table(utT_hbm, uflat_hbm, cnt_u)
    cnt_i = build_wl(items_hbm)
    bucketize(cnt_i)
    pass_table(itT_hbm, iflat_hbm, cnt_i)


@functools.partial(
    pl.kernel,
    out_type=jax.ShapeDtypeStruct((B,), jnp.float32),
    mesh=_mesh,
    compiler_params=pltpu.CompilerParams(needs_layout_passes=False),
    scratch_types=[
        pltpu.VMEM((BPW * D,), jnp.float32),
        pltpu.VMEM((BPW * D,), jnp.float32),
        pltpu.VMEM((BPW,), jnp.float32),
        pltpu.SemaphoreType.DMA,
        pltpu.SemaphoreType.DMA,
    ],
)
def _mf_dot(uflat_hbm, iflat_hbm, out_hbm, uv, iv, out_v, semU, semI):
    wid = lax.axis_index("s") * 2 + lax.axis_index("c")
    base = wid * BPW
    cu = pltpu.async_copy(uflat_hbm.at[pl.ds(base * D, BPW * D)], uv, semU)
    ci = pltpu.async_copy(iflat_hbm.at[pl.ds(base * D, BPW * D)], iv, semI)
    cu.wait()
    ci.wait()
    for b0 in range(BPW // L):
        acc = jnp.zeros((L,), jnp.float32)
        row = (b0 * L + _iota()) * D

        def body(d, acc):
            idx = row + d
            u = plsc.load_gather(uv, [idx])
            i = plsc.load_gather(iv, [idx])
            return acc + u * i

        acc = lax.fori_loop(0, D, body, acc)
        out_v[pl.ds(b0 * L, L)] = acc
    pltpu.sync_copy(out_v, out_hbm.at[pl.ds(base, BPW)])


def kernel(users, items, user_table, item_table):
    u = users.astype(jnp.int32)
    i = items.astype(jnp.int32)
    u_flat, i_flat = _mf_stream_extract(u, i, user_table.T, item_table.T)
    return _mf_dot(u_flat, i_flat)


# speedup vs baseline: 1.5028x; 1.1017x over previous
import functools

import jax
import jax.numpy as jnp
from jax import lax
from jax.experimental import pallas as pl
from jax.experimental.pallas import tpu as pltpu
from jax.experimental.pallas import tpu_sc as plsc

L = 16
NW = 32
B = 16384
D = 64
BPW = B // NW
V = 1000000
BLK = 128
NBLK = (V + BLK - 1) // BLK
PHYS_COLS = NBLK * BLK
RB = 2
RW = RB * BLK
RSH = 8
NR = 123
NRP = 128
NBUF = 4
LAST_BASE = PHYS_COLS - RW
SR = 16
WLSZ = B + L
SENT = 1 << 29

_mesh = plsc.VectorSubcoreMesh(core_axis_name="c", subcore_axis_name="s")


def _iota():
    return lax.broadcasted_iota(jnp.int32, (L,), 0)


def _lane(vec, lane):
    idx = jnp.full((L,), lane, jnp.int32)
    dnums = lax.GatherDimensionNumbers(
        offset_dims=(), collapsed_slice_dims=(0,), start_index_map=(0,))
    g = lax.gather(vec, idx[:, None], dnums, (1,),
                   mode=lax.GatherScatterMode.PROMISE_IN_BOUNDS)
    return g[0]


@functools.partial(
    pl.kernel,
    out_type=(
        jax.ShapeDtypeStruct((B * D,), jnp.float32),
        jax.ShapeDtypeStruct((B * D,), jnp.float32),
    ),
    mesh=_mesh,
    compiler_params=pltpu.CompilerParams(needs_layout_passes=False),
    scratch_types=[
        pltpu.VMEM((2048,), jnp.int32),
        pltpu.VMEM((WLSZ,), jnp.int32),
        pltpu.VMEM((WLSZ,), jnp.int32),
        pltpu.VMEM((NRP,), jnp.int32),
        pltpu.VMEM((NRP,), jnp.int32),
        pltpu.VMEM((NRP,), jnp.int32),
        pltpu.VMEM((D, RW), jnp.float32),
        pltpu.VMEM((D, RW), jnp.float32),
        pltpu.VMEM((D, RW), jnp.float32),
        pltpu.VMEM((D, RW), jnp.float32),
        pltpu.VMEM((SR, D), jnp.float32),
        pltpu.SemaphoreType.DMA,
        pltpu.SemaphoreType.DMA,
        pltpu.SemaphoreType.DMA,
        pltpu.SemaphoreType.DMA,
        pltpu.SemaphoreType.DMA,
        pltpu.SemaphoreType.DMA,
    ],
)
def _mf_stream_extract(users_hbm, items_hbm, utT_hbm, itT_hbm,
                       uflat_hbm, iflat_hbm,
                       chunk, wl, wl2, counts, starts, cur,
                       buf0, buf1, buf2, buf3, stage,
                       sem0, sem1, sem2, sem3, semX, semC):
    bufs = (buf0, buf1, buf2, buf3)
    sems = (sem0, sem1, sem2, sem3)

    def drain_one(i, carry):
        pltpu.make_async_copy(
            uflat_hbm.at[pl.ds(0, D)], stage.at[0], semX).wait()
        return carry
    wid = lax.axis_index("s") * 2 + lax.axis_index("c")
    c0 = 244 * wid + jnp.minimum(wid, 5)
    nb = jnp.where(wid < 5, 245, 244)
    lo_k = c0 * BLK
    hi_k = (c0 + nb) * BLK

    def build_wl(src_hbm):
        for i in range(NRP // L):
            counts[pl.ds(i * L, L)] = jnp.zeros((L,), jnp.int32)

        def chunk_body(s, cnt):
            pltpu.async_copy(src_hbm.at[pl.ds(s * 2048, 2048)], chunk,
                             semC).wait()
            def vec_body(v, cnt):
                kv = chunk[pl.ds(v * L, L)]
                bv = (s * 2048 + v * L) + _iota()
                m = (kv >= lo_k) & (kv < hi_k)
                pack = ((kv - lo_k) << 14) | bv
                cs = plsc.cumsum(m.astype(jnp.int32))
                pos = jnp.maximum(cnt + cs - 1, 0)
                plsc.store_scatter(wl, [pos], pack, mask=m)
                rv = jnp.clip((kv - lo_k) >> RSH, 0, NRP - 1)
                plsc.addupdate_scatter(counts, [rv],
                                       jnp.ones((L,), jnp.int32), mask=m)
                return cnt + cs[L - 1]
            return lax.fori_loop(0, 2048 // L, vec_body, cnt)
        return lax.fori_loop(0, B // 2048, chunk_body, jnp.int32(0))

    def bucketize(cnt):
        carry = jnp.int32(0)
        for i in range(NRP // L):
            c = counts[pl.ds(i * L, L)]
            cs = plsc.cumsum(c)
            sv = cs - c + carry
            starts[pl.ds(i * L, L)] = sv
            cur[pl.ds(i * L, L)] = sv
            carry = carry + cs[L - 1]

        def place(j, _):
            mv = (j * L + _iota()) < cnt
            wv = wl[pl.ds(j * L, L)]
            rv = jnp.clip((wv >> 14) >> RSH, 0, NRP - 1)
            dup, _last = plsc.scan_count(rv, mask=mv)
            basev = plsc.load_gather(cur, [rv])
            plsc.store_scatter(wl2, [basev + dup - 1], wv, mask=mv)
            plsc.addupdate_scatter(cur, [rv],
                                   jnp.ones((L,), jnp.int32), mask=mv)
            return 0
        lax.fori_loop(0, (cnt + L - 1) // L, place, 0)

    def issue(src_hbm, r, buf, sem):
        base_k = jnp.minimum((c0 + RB * r) * BLK, LAST_BASE)
        base_k = pl.multiple_of(base_k, BLK)
        return pltpu.async_copy(src_hbm.at[:, pl.ds(base_k, RW)], buf, sem)

    def pass_table(src_hbm, dst_hbm, cnt):
        def process_round(r, buf, sem, h):
            pltpu.make_async_copy(src_hbm.at[:, pl.ds(0, RW)], buf, sem).wait()
            r_lo = (c0 + RB * r) * BLK
            base_k = jnp.minimum(r_lo, LAST_BASE)
            g = plsc.load_gather(starts, [r + jnp.minimum(_iota(), 1)])
            s_r = g[0]
            e_r = g[1]

            def seg_cond(state):
                t, _ = state
                return t < e_r

            def seg_body(state):
                t, h = state
                wv = plsc.load_gather(wl2, [t + _iota()])
                m = (t + _iota()) < e_r

                def hit_cond(state):
                    m, _ = state
                    return jnp.any(m)

                def hit_body(state):
                    m, h = state
                    l_vec = plsc.all_reduce_ffs(m)
                    pk = _lane(wv, l_vec[0])
                    kg_s = (pk >> 14) + lo_k
                    b_s = pk & (B - 1)
                    colw = kg_s - base_k
                    hmod = h % SR
                    cvec = jnp.full((L,), colw, jnp.int32)
                    for dc in range(D // L):
                        g = plsc.load_gather(
                            buf, [dc * L + _iota(), cvec])
                        stage[hmod, pl.ds(dc * L, L)] = g
                    off = pl.multiple_of(b_s * D, 8)
                    pltpu.async_copy(stage.at[hmod],
                                     dst_hbm.at[pl.ds(off, D)], semX)
                    @pl.when((h + 1) % SR == 0)
                    def _():
                        lax.fori_loop(0, SR, drain_one, 0)
                    return m & (_iota() != l_vec), h + 1

                m, h = lax.while_loop(hit_cond, hit_body, (m, h))
                return t + L, h

            _, h = lax.while_loop(seg_cond, seg_body, (s_r, h))
            return h

        for par in range(NBUF):
            issue(src_hbm, par, bufs[par], sems[par])

        def group_body(rr, h):
            for par in range(NBUF):
                r = rr * NBUF + par
                h = lax.cond(
                    r < NR,
                    lambda h, r=r, par=par: process_round(
                        r, bufs[par], sems[par], h),
                    lambda h: h,
                    h,
                )

                @pl.when(r + NBUF < NR)
                def _(r=r, par=par):
                    issue(src_hbm, r + NBUF, bufs[par], sems[par])
            return h

        h = lax.fori_loop(0, (NR + NBUF - 1) // NBUF, group_body, jnp.int32(0))
        lax.fori_loop(0, h % SR, drain_one, 0)

    cnt_u = build_wl(users_hbm)
    bucketize(cnt_u)
    pass_table(utT_hbm, uflat_hbm, cnt_u)
    cnt_i = build_wl(items_hbm)
    bucketize(cnt_i)
    pass_table(itT_hbm, iflat_hbm, cnt_i)


@functools.partial(
    pl.kernel,
    out_type=jax.ShapeDtypeStruct((B,), jnp.float32),
    mesh=_mesh,
    compiler_params=pltpu.CompilerParams(needs_layout_passes=False),
    scratch_types=[
        pltpu.VMEM((BPW * D,), jnp.float32),
        pltpu.VMEM((BPW * D,), jnp.float32),
        pltpu.VMEM((BPW,), jnp.float32),
        pltpu.SemaphoreType.DMA,
        pltpu.SemaphoreType.DMA,
    ],
)
def _mf_dot(uflat_hbm, iflat_hbm, out_hbm, uv, iv, out_v, semU, semI):
    wid = lax.axis_index("s") * 2 + lax.axis_index("c")
    base = wid * BPW
    cu = pltpu.async_copy(uflat_hbm.at[pl.ds(base * D, BPW * D)], uv, semU)
    ci = pltpu.async_copy(iflat_hbm.at[pl.ds(base * D, BPW * D)], iv, semI)
    cu.wait()
    ci.wait()
    def b_body(b, _):
        acc = jnp.zeros((L,), jnp.float32)
        for j in range(D // L):
            u = uv[pl.ds(b * D + j * L, L)]
            i = iv[pl.ds(b * D + j * L, L)]
            acc = acc + u * i
        cs = plsc.cumsum(acc)
        plsc.store_scatter(out_v, [jnp.full((L,), b, jnp.int32)],
                           jnp.full((L,), cs[L - 1], jnp.float32),
                           mask=_iota() == 0)
        return 0

    lax.fori_loop(0, BPW, b_body, 0)
    pltpu.sync_copy(out_v, out_hbm.at[pl.ds(base, BPW)])


def kernel(users, items, user_table, item_table):
    u = users.astype(jnp.int32)
    i = items.astype(jnp.int32)
    u_flat, i_flat = _mf_stream_extract(u, i, user_table.T, item_table.T)
    return _mf_dot(u_flat, i_flat)
